# baseline (speedup 1.0000x reference)
import functools

import jax
import jax.numpy as jnp
from jax import lax
from jax.experimental import pallas as pl
from jax.experimental.pallas import tpu as pltpu
from jax.experimental.pallas import tpu_sc as plsc

HID = 128
CDIM = 128
LAT = 64
MER = 64
NCH = 4
T = 20000
B = 2
TP = 2000
NT = T // TP
NCELL = LAT * MER

_CENTERS = ((0.25, 0.25, 0.25), (-0.25, -0.25, -0.25),
            (-0.25, 0.25, 0.25), (0.25, -0.25, 0.25))
_PI = 3.1415927410125732


def _block0_body(p_ref, w1, b1, w0, b0, wf1, bf1, ws, net_out, max_out):
    t = pl.program_id(1)
    x = p_ref[0]
    net = jnp.dot(x, w1[...], preferred_element_type=jnp.float32) + b1[...]
    h = jnp.dot(jnp.maximum(net, 0.0), w0[...],
                preferred_element_type=jnp.float32) + b0[...]
    dx = jnp.dot(jnp.maximum(h, 0.0), wf1[...],
                 preferred_element_type=jnp.float32) + bf1[...]
    out = jnp.dot(net, ws[...], preferred_element_type=jnp.float32) + dx
    net_out[0] = out
    tile_max = jnp.max(out, axis=0, keepdims=True)
    prev = jnp.where(t == 0, jnp.full((1, HID), -jnp.inf, jnp.float32),
                     max_out[0])
    max_out[0] = jnp.maximum(prev, tile_max)


def _blockn_body(net_ref, pool_ref, w0, b0, wf1, bf1, ws, net_out, max_out):
    t = pl.program_id(1)
    x1 = net_ref[0]
    pooled = jnp.broadcast_to(pool_ref[0], x1.shape)
    x = jnp.concatenate([x1, pooled], axis=1)
    h = jnp.dot(jnp.maximum(x, 0.0), w0[...],
                preferred_element_type=jnp.float32) + b0[...]
    dx = jnp.dot(jnp.maximum(h, 0.0), wf1[...],
                 preferred_element_type=jnp.float32) + bf1[...]
    out = jnp.dot(x, ws[...], preferred_element_type=jnp.float32) + dx
    net_out[0] = out
    if max_out is not None:
        tile_max = jnp.max(out, axis=0, keepdims=True)
        prev = jnp.where(t == 0, jnp.full((1, HID), -jnp.inf, jnp.float32),
                         max_out[0])
        max_out[0] = jnp.maximum(prev, tile_max)


def _last_body(nr, pr, w0, b0, wf1, bf1, ws, no):
    _blockn_body(nr, pr, w0, b0, wf1, bf1, ws, no, None)


def _wspec(shape):
    return pl.BlockSpec(shape, lambda b, t: (0,) * len(shape))


def _pointnet(p, P):
    grid = (B, NT)
    net_spec = pl.BlockSpec((1, TP, HID), lambda b, t: (b, t, 0))
    max_spec = pl.BlockSpec((1, 1, HID), lambda b, t: (b, 0, 0))
    net0, max0 = pl.pallas_call(
        _block0_body,
        grid=grid,
        in_specs=[
            pl.BlockSpec((1, TP, 3), lambda b, t: (b, t, 0)),
            _wspec((3, 2 * HID)), _wspec((1, 2 * HID)),
            _wspec((2 * HID, HID)), _wspec((1, HID)),
            _wspec((HID, HID)), _wspec((1, HID)),
            _wspec((2 * HID, HID)),
        ],
        out_specs=[net_spec, max_spec],
        out_shape=[jax.ShapeDtypeStruct((B, T, HID), jnp.float32),
                   jax.ShapeDtypeStruct((B, 1, HID), jnp.float32)],
    )(p, P['fc_pos_W'], P['fc_pos_b'][None],
      P['b0_fc0_W'], P['b0_fc0_b'][None],
      P['b0_fc1_W'], P['b0_fc1_b'][None], P['b0_sc_W'])

    net, mx = net0, max0
    for b in range(1, 5):
        last = b == 4
        outs = [jax.ShapeDtypeStruct((B, T, HID), jnp.float32)]
        ospecs = [net_spec]
        if not last:
            outs.append(jax.ShapeDtypeStruct((B, 1, HID), jnp.float32))
            ospecs.append(max_spec)
        res = pl.pallas_call(
            _last_body if last else _blockn_body,
            grid=grid,
            in_specs=[
                net_spec, max_spec,
                _wspec((2 * HID, HID)), _wspec((1, HID)),
                _wspec((HID, HID)), _wspec((1, HID)),
                _wspec((2 * HID, HID)),
            ],
            out_specs=ospecs,
            out_shape=outs,
        )(net, mx, P['b%d_fc0_W' % b], P['b%d_fc0_b' % b][None],
          P['b%d_fc1_W' % b], P['b%d_fc1_b' % b][None], P['b%d_sc_W' % b])
        if last:
            net = res[0]
        else:
            net, mx = res
    return net


def _bin_body(p_ref, cell_ref, rad_ref):
    x = p_ref[:, :, 0]
    y = p_ref[:, :, 1]
    z = p_ref[:, :, 2]
    rad_ref[0] = jnp.sqrt(x * x + y * y + z * z)
    dim_merid = 360.0 / MER
    dim_lat = 180.0 / LAT
    for l in range(NCH):
        cx, cy, cz = _CENTERS[l]
        xv = x - cx
        yv = y - cy
        zv = z - cz
        lati = 90.0 - jnp.arctan2(zv, jnp.sqrt(xv * xv + yv * yv)) * 180.0 / _PI
        meri = (360.0 + jnp.arctan2(yv, xv) * 180.0 / _PI) % 360.0
        y_grid = jnp.floor(lati / dim_lat)
        x_grid = jnp.floor(meri / dim_merid)
        cell = (x_grid + MER * y_grid).astype(jnp.int32)
        for b in range(B):
            cell_ref[0, b * NCH + l] = cell[b]


def _binning(p):
    cell, rad = pl.pallas_call(
        _bin_body,
        grid=(NT,),
        in_specs=[pl.BlockSpec((B, TP, 3), lambda t: (0, t, 0))],
        out_specs=[pl.BlockSpec((1, B * NCH, TP), lambda t: (t, 0, 0)),
                   pl.BlockSpec((1, B, TP), lambda t: (t, 0, 0))],
        out_shape=[jax.ShapeDtypeStruct((NT, B * NCH, TP), jnp.int32),
                   jax.ShapeDtypeStruct((NT, B, TP), jnp.float32)],
    )(p)
    return cell, rad


_SC_MESH = dict(core_axis_name="c", subcore_axis_name="s",
                num_cores=2, num_subcores=16)
OCAP = 2080
CPTS = 1024
NWAVE = 2
CPT = NCELL // (4 * NWAVE)
GC = 32


def _sc_stats_body(cell_hbm, rad_hbm, hmean_hbm, hmax_hbm,
                   cnt_a, sum_a, max_a, cells, rads, orel, orval,
                   hmean_s, hmax_s):
    wid = lax.axis_index("s") * 2 + lax.axis_index("c")
    tl = wid // 4
    sub = wid % 4
    b = tl // NCH
    iota16 = lax.iota(jnp.int32, 16)
    zero16 = jnp.zeros((16,), jnp.float32)
    lane0 = iota16 == 0

    def zst(i, _):
        cnt_a[pl.ds(i * 16, 16)] = zero16
        sum_a[pl.ds(i * 16, 16)] = zero16
        max_a[pl.ds(i * 16, 16)] = zero16
        return 0
    lax.fori_loop(0, (CPTS + 16) // 16, zst, 0)

    def chunk_body(m, _):
        pltpu.sync_copy(cell_hbm.at[pl.ds((m * 8 + tl) * TP, TP)], cells)
        pltpu.sync_copy(rad_hbm.at[pl.ds((m * 2 + b) * TP, TP)], rads)

        def comp(j, cur):
            v = cells[pl.ds(j * 16, 16)]
            d = v - sub
            msk = (d >= 0) & (d < NCELL) & ((d & 3) == 0)
            rel = lax.shift_right_logical(d, 2)
            pc = plsc.cumsum(msk.astype(jnp.int32))
            dst = cur + pc - 1
            plsc.store_scatter(orel, [dst], rel, mask=msk)
            plsc.store_scatter(orval, [dst], rads[pl.ds(j * 16, 16)],
                               mask=msk)
            return cur + pc[15]
        cnt = lax.fori_loop(0, TP // 16, comp, 0)

        def pt_body(i, _):
            lc = orel[pl.ds(i, 16)][0]
            rv = orval[pl.ds(i, 16)][0]
            rvv = jnp.full((16,), rv, jnp.float32)
            c = cnt_a[pl.ds(lc, 16)]
            cnt_a[pl.ds(lc, 16)] = c + jnp.where(lane0, jnp.float32(1.0),
                                                 jnp.float32(0.0))
            sm = sum_a[pl.ds(lc, 16)]
            sum_a[pl.ds(lc, 16)] = sm + jnp.where(lane0, rvv,
                                                  jnp.float32(0.0))
            mx = max_a[pl.ds(lc, 16)]
            max_a[pl.ds(lc, 16)] = jnp.maximum(
                mx, jnp.where(lane0, rvv, jnp.full((16,), -jnp.inf,
                                                   jnp.float32)))
            return 0
        lax.fori_loop(0, cnt, pt_body, 0)
        return 0
    lax.fori_loop(0, NT, chunk_body, 0)

    def ep(g, _):
        cnt16 = cnt_a[pl.ds(g * 16, 16)]
        sum16 = sum_a[pl.ds(g * 16, 16)]
        max16 = max_a[pl.ds(g * 16, 16)]
        hmean_s[pl.ds(g * 16, 16)] = sum16 / jnp.maximum(cnt16, 1.0)
        hmax_s[pl.ds(g * 16, 16)] = max16
        return 0
    lax.fori_loop(0, CPTS // 16, ep, 0)

    pltpu.sync_copy(hmean_s, hmean_hbm.at[pl.ds((tl * 4 + sub) * CPTS, CPTS)])
    pltpu.sync_copy(hmax_s, hmax_hbm.at[pl.ds((tl * 4 + sub) * CPTS, CPTS)])


def _sc_nmax_body(cell_hbm, net_hbm, nmax_hbm,
                  acc, cells, oidx, orel, rows, sem):
    wid = lax.axis_index("s") * 2 + lax.axis_index("c")
    tl = wid // 4
    sub = wid % 4
    b = tl // NCH
    iota16 = lax.iota(jnp.int32, 16)
    zero16 = jnp.zeros((16,), jnp.float32)
    zero16i = jnp.zeros((16,), jnp.int32)
    for w in range(NWAVE):

        def zacc(i, _):
            acc[pl.ds(i * 16, 16)] = zero16
            return 0
        lax.fori_loop(0, CPT * HID // 16, zacc, 0)

        def chunk_body(m, _):
            pltpu.sync_copy(cell_hbm.at[pl.ds((m * 8 + tl) * TP, TP)], cells)
            base = b * T + m * TP

            def comp(j, cur):
                v = cells[pl.ds(j * 16, 16)]
                d = v - (w * 4 * CPT + sub)
                msk = (d >= 0) & (d < 4 * CPT) & ((d & 3) == 0)
                rel = lax.shift_right_logical(d, 2)
                pc = plsc.cumsum(msk.astype(jnp.int32))
                dst = cur + pc - 1
                plsc.store_scatter(oidx, [dst],
                                   iota16 + (base + j * 16), mask=msk)
                plsc.store_scatter(orel, [dst], rel, mask=msk)
                return cur + pc[15]
            cnt = lax.fori_loop(0, TP // 16, comp, 0)

            for j in range(GC // 16):
                plsc.store_scatter(oidx, [cnt + j * 16 + iota16],
                                   zero16i + base)

            nsub = (cnt + GC - 1) // GC

            def fire(k):
                pltpu.async_copy(
                    net_hbm.at[oidx.at[pl.ds(k * GC, GC)]],
                    rows.at[pl.ds((k % 2) * GC, GC)], sem)

            def drain(k):
                pltpu.make_async_copy(
                    net_hbm.at[oidx.at[pl.ds(k * GC, GC)]],
                    rows.at[pl.ds((k % 2) * GC, GC)], sem).wait()

            @pl.when(nsub > 0)
            def _():
                fire(0)

            def sub_body(k, _):
                @pl.when(k + 1 < nsub)
                def _():
                    fire(k + 1)
                drain(k)
                npts = jnp.minimum(GC, cnt - k * GC)
                rb = (k % 2) * GC

                def pt_body(i, _):
                    lc = orel[pl.ds(k * GC + i, 16)][0]
                    off = lc * HID
                    for f in range(HID // 16):
                        a = acc[pl.ds(off + f * 16, 16)]
                        r = rows[rb + i, pl.ds(f * 16, 16)]
                        acc[pl.ds(off + f * 16, 16)] = jnp.maximum(a, r)
                    return 0
                lax.fori_loop(0, npts, pt_body, 0)
                return 0
            lax.fori_loop(0, nsub, sub_body, 0)
            return 0
        lax.fori_loop(0, NT, chunk_body, 0)

        pltpu.sync_copy(
            acc,
            nmax_hbm.at[pl.ds((((tl * NWAVE + w) * 4 + sub) * CPT) * HID,
                              CPT * HID)])


def _scatter(cell3, rad3, net):
    netflat = net.reshape(B * T, HID)
    cellf = cell3.reshape(NT * B * NCH * TP)
    radf = rad3.reshape(NT * B * TP)
    stats_kern = pl.kernel(
        _sc_stats_body,
        out_type=[
            jax.ShapeDtypeStruct((B * NCH * NCELL,), jnp.float32),
            jax.ShapeDtypeStruct((B * NCH * NCELL,), jnp.float32),
        ],
        mesh=plsc.VectorSubcoreMesh(**_SC_MESH),
        compiler_params=pltpu.CompilerParams(needs_layout_passes=False),
        scratch_types=[
            pltpu.VMEM((CPTS + 16,), jnp.float32),
            pltpu.VMEM((CPTS + 16,), jnp.float32),
            pltpu.VMEM((CPTS + 16,), jnp.float32),
            pltpu.VMEM((TP,), jnp.int32),
            pltpu.VMEM((TP,), jnp.float32),
            pltpu.VMEM((OCAP,), jnp.int32),
            pltpu.VMEM((OCAP,), jnp.float32),
            pltpu.VMEM((CPTS,), jnp.float32),
            pltpu.VMEM((CPTS,), jnp.float32),
        ],
    )
    hmean, hmax = stats_kern(cellf, radf)
    nmax_kern = pl.kernel(
        _sc_nmax_body,
        out_type=jax.ShapeDtypeStruct((B * NCH * NCELL * HID,), jnp.float32),
        mesh=plsc.VectorSubcoreMesh(**_SC_MESH),
        compiler_params=pltpu.CompilerParams(needs_layout_passes=False),
        scratch_types=[
            pltpu.VMEM((CPT * HID,), jnp.float32),
            pltpu.VMEM((TP,), jnp.int32),
            pltpu.VMEM((OCAP,), jnp.int32),
            pltpu.VMEM((OCAP,), jnp.int32),
            pltpu.VMEM((2 * GC, HID), jnp.float32),
            pltpu.SemaphoreType.DMA,
        ],
    )
    nmax = nmax_kern(cellf, netflat)
    nmax = (nmax.reshape(B * NCH, NWAVE, 4, CPT, HID)
            .transpose(0, 1, 3, 2, 4).reshape(B * NCH, NCELL, HID))
    hmean = (hmean.reshape(B * NCH, 4, CPTS)
             .transpose(0, 2, 1).reshape(B * NCH, NCELL))
    hmax = (hmax.reshape(B * NCH, 4, CPTS)
            .transpose(0, 2, 1).reshape(B * NCH, NCELL))
    return nmax, hmean, hmax


def _conv3(xp_ref, x, w_ref, b_ref, H, W, relu=True):
    Cin = x.shape[-1]
    Co = w_ref.shape[-1]
    xp_ref[...] = jnp.zeros((H + 2, W + 2, Cin), jnp.float32)
    xp_ref[1:H + 1, 1:W + 1, :] = x
    acc = jnp.broadcast_to(b_ref[0], (H * W, Co))
    for dh in range(3):
        for dw in range(3):
            patch = xp_ref[dh:dh + H, dw:dw + W, :].reshape(H * W, Cin)
            acc = acc + jnp.dot(patch, w_ref[dh, dw],
                                preferred_element_type=jnp.float32)
    out = acc.reshape(H, W, Co)
    return jnp.maximum(out, 0.0) if relu else out


def _unet_body(f_ref, w1, b1, w2, b2, w3, b3, w4, b4, w5, b5, w6, b6,
               out_ref, xp1, xp2, xp3, xp4, xp5):
    x = f_ref[0]
    h = _conv3(xp1, x, w1, b1, 64, 64)
    h = _conv3(xp2, h, w2, b2, 64, 64)
    skip = h
    d = jnp.max(h.reshape(32, 2, 64, 64), axis=1)
    d = jnp.max(d.reshape(32, 32, 2, 64), axis=2)
    d = _conv3(xp3, d, w3, b3, 32, 32)
    d = _conv3(xp4, d, w4, b4, 32, 32)
    u = jnp.broadcast_to(d.reshape(32, 1, 32, 128),
                         (32, 2, 32, 128)).reshape(64, 32, 128)
    u = jnp.broadcast_to(u.reshape(64, 32, 1, 128),
                         (64, 32, 2, 128)).reshape(64, 64, 128)
    u = jnp.concatenate([u, skip], axis=2)
    u = _conv3(xp5, u, w5, b5, 64, 64)
    out = jnp.dot(u.reshape(64 * 64, 64), w6[0, 0],
                  preferred_element_type=jnp.float32) + b6[0]
    out_ref[0] = out.reshape(64, 64, CDIM)


def _unet(feat, P):
    def ws(shape):
        return pl.BlockSpec(shape, lambda i: (0,) * len(shape))
    return pl.pallas_call(
        _unet_body,
        grid=(B * NCH,),
        in_specs=[
            pl.BlockSpec((1, 64, 64, HID + 3), lambda i: (i, 0, 0, 0)),
            ws((3, 3, HID + 3, 64)), ws((1, 64)),
            ws((3, 3, 64, 64)), ws((1, 64)),
            ws((3, 3, 64, 128)), ws((1, 128)),
            ws((3, 3, 128, 128)), ws((1, 128)),
            ws((3, 3, 192, 64)), ws((1, 64)),
            ws((1, 1, 64, CDIM)), ws((1, CDIM)),
        ],
        out_specs=pl.BlockSpec((1, 64, 64, CDIM), lambda i: (i, 0, 0, 0)),
        out_shape=jax.ShapeDtypeStruct((B * NCH, 64, 64, CDIM), jnp.float32),
        scratch_shapes=[
            pltpu.VMEM((66, 66, HID + 3), jnp.float32),
            pltpu.VMEM((66, 66, 64), jnp.float32),
            pltpu.VMEM((34, 34, 64), jnp.float32),
            pltpu.VMEM((34, 34, 128), jnp.float32),
            pltpu.VMEM((66, 66, 192), jnp.float32),
        ],
    )(feat, P['u1_W'], P['u1_b'][None], P['u2_W'], P['u2_b'][None],
      P['u3_W'], P['u3_b'][None], P['u4_W'], P['u4_b'][None],
      P['u5_W'], P['u5_b'][None], P['u6_W'], P['u6_b'][None])


def kernel(p, params):
    net = _pointnet(p, params)
    cell, rad = _binning(p)
    nmax, hmean, hmax = _scatter(cell, rad, net)
    feat = jnp.concatenate([
        nmax,
        jnp.zeros((B * NCH, NCELL, 1), jnp.float32),
        hmean[..., None],
        hmax[..., None],
    ], axis=2).reshape(B * NCH, LAT, MER, HID + 3)
    c = _unet(feat, params)
    c = c.reshape(B, NCH, LAT, MER, CDIM)
    centers = jnp.array(_CENTERS, dtype=jnp.float32)
    sphere_centers = jnp.broadcast_to(centers[None], (B, NCH, 3))
    return c, sphere_centers


# speedup vs baseline: 1.1060x; 1.1060x over previous
import functools

import jax
import jax.numpy as jnp
from jax import lax
from jax.experimental import pallas as pl
from jax.experimental.pallas import tpu as pltpu
from jax.experimental.pallas import tpu_sc as plsc

HID = 128
CDIM = 128
LAT = 64
MER = 64
NCH = 4
T = 20000
B = 2
TP = 2000
NT = T // TP
NCELL = LAT * MER

_CENTERS = ((0.25, 0.25, 0.25), (-0.25, -0.25, -0.25),
            (-0.25, 0.25, 0.25), (0.25, -0.25, 0.25))
_PI = 3.1415927410125732


def _block0_body(p_ref, w1, b1, w0, b0, wf1, bf1, ws, net_out, max_out):
    t = pl.program_id(1)
    x = p_ref[0]
    net = jnp.dot(x, w1[...], preferred_element_type=jnp.float32) + b1[...]
    h = jnp.dot(jnp.maximum(net, 0.0), w0[...],
                preferred_element_type=jnp.float32) + b0[...]
    dx = jnp.dot(jnp.maximum(h, 0.0), wf1[...],
                 preferred_element_type=jnp.float32) + bf1[...]
    out = jnp.dot(net, ws[...], preferred_element_type=jnp.float32) + dx
    net_out[0] = out
    tile_max = jnp.max(out, axis=0, keepdims=True)
    prev = jnp.where(t == 0, jnp.full((1, HID), -jnp.inf, jnp.float32),
                     max_out[0])
    max_out[0] = jnp.maximum(prev, tile_max)


def _blockn_body(net_ref, pool_ref, w0, b0, wf1, bf1, ws, net_out, max_out):
    t = pl.program_id(1)
    x1 = net_ref[0]
    pooled = jnp.broadcast_to(pool_ref[0], x1.shape)
    x = jnp.concatenate([x1, pooled], axis=1)
    h = jnp.dot(jnp.maximum(x, 0.0), w0[...],
                preferred_element_type=jnp.float32) + b0[...]
    dx = jnp.dot(jnp.maximum(h, 0.0), wf1[...],
                 preferred_element_type=jnp.float32) + bf1[...]
    out = jnp.dot(x, ws[...], preferred_element_type=jnp.float32) + dx
    net_out[0] = out
    if max_out is not None:
        tile_max = jnp.max(out, axis=0, keepdims=True)
        prev = jnp.where(t == 0, jnp.full((1, HID), -jnp.inf, jnp.float32),
                         max_out[0])
        max_out[0] = jnp.maximum(prev, tile_max)


def _last_body(nr, pr, w0, b0, wf1, bf1, ws, no):
    _blockn_body(nr, pr, w0, b0, wf1, bf1, ws, no, None)


def _wspec(shape):
    return pl.BlockSpec(shape, lambda b, t: (0,) * len(shape))


def _pointnet(p, P):
    grid = (B, NT)
    net_spec = pl.BlockSpec((1, TP, HID), lambda b, t: (b, t, 0))
    max_spec = pl.BlockSpec((1, 1, HID), lambda b, t: (b, 0, 0))
    net0, max0 = pl.pallas_call(
        _block0_body,
        grid=grid,
        in_specs=[
            pl.BlockSpec((1, TP, 3), lambda b, t: (b, t, 0)),
            _wspec((3, 2 * HID)), _wspec((1, 2 * HID)),
            _wspec((2 * HID, HID)), _wspec((1, HID)),
            _wspec((HID, HID)), _wspec((1, HID)),
            _wspec((2 * HID, HID)),
        ],
        out_specs=[net_spec, max_spec],
        out_shape=[jax.ShapeDtypeStruct((B, T, HID), jnp.float32),
                   jax.ShapeDtypeStruct((B, 1, HID), jnp.float32)],
    )(p, P['fc_pos_W'], P['fc_pos_b'][None],
      P['b0_fc0_W'], P['b0_fc0_b'][None],
      P['b0_fc1_W'], P['b0_fc1_b'][None], P['b0_sc_W'])

    net, mx = net0, max0
    for b in range(1, 5):
        last = b == 4
        outs = [jax.ShapeDtypeStruct((B, T, HID), jnp.float32)]
        ospecs = [net_spec]
        if not last:
            outs.append(jax.ShapeDtypeStruct((B, 1, HID), jnp.float32))
            ospecs.append(max_spec)
        res = pl.pallas_call(
            _last_body if last else _blockn_body,
            grid=grid,
            in_specs=[
                net_spec, max_spec,
                _wspec((2 * HID, HID)), _wspec((1, HID)),
                _wspec((HID, HID)), _wspec((1, HID)),
                _wspec((2 * HID, HID)),
            ],
            out_specs=ospecs,
            out_shape=outs,
        )(net, mx, P['b%d_fc0_W' % b], P['b%d_fc0_b' % b][None],
          P['b%d_fc1_W' % b], P['b%d_fc1_b' % b][None], P['b%d_sc_W' % b])
        if last:
            net = res[0]
        else:
            net, mx = res
    return net


def _bin_body(p_ref, cell_ref, rad_ref):
    x = p_ref[:, :, 0]
    y = p_ref[:, :, 1]
    z = p_ref[:, :, 2]
    rad_ref[0] = jnp.sqrt(x * x + y * y + z * z)
    dim_merid = 360.0 / MER
    dim_lat = 180.0 / LAT
    for l in range(NCH):
        cx, cy, cz = _CENTERS[l]
        xv = x - cx
        yv = y - cy
        zv = z - cz
        lati = 90.0 - jnp.arctan2(zv, jnp.sqrt(xv * xv + yv * yv)) * 180.0 / _PI
        meri = (360.0 + jnp.arctan2(yv, xv) * 180.0 / _PI) % 360.0
        y_grid = jnp.floor(lati / dim_lat)
        x_grid = jnp.floor(meri / dim_merid)
        cell = (x_grid + MER * y_grid).astype(jnp.int32)
        for b in range(B):
            cell_ref[0, b * NCH + l] = cell[b]


def _binning(p):
    cell, rad = pl.pallas_call(
        _bin_body,
        grid=(NT,),
        in_specs=[pl.BlockSpec((B, TP, 3), lambda t: (0, t, 0))],
        out_specs=[pl.BlockSpec((1, B * NCH, TP), lambda t: (t, 0, 0)),
                   pl.BlockSpec((1, B, TP), lambda t: (t, 0, 0))],
        out_shape=[jax.ShapeDtypeStruct((NT, B * NCH, TP), jnp.int32),
                   jax.ShapeDtypeStruct((NT, B, TP), jnp.float32)],
    )(p)
    return cell, rad


_SC_MESH = dict(core_axis_name="c", subcore_axis_name="s",
                num_cores=2, num_subcores=16)
OCAP = 2080
CPTS = 1024
NWAVE = 2
CPT = NCELL // (4 * NWAVE)
GC = 32


def _sc_stats_body(cell_hbm, rad_hbm, hmean_hbm, hmax_hbm,
                   cnt_a, sum_a, max_a, cells, rads, orel, orval,
                   hmean_s, hmax_s):
    wid = lax.axis_index("s") * 2 + lax.axis_index("c")
    tl = wid // 4
    sub = wid % 4
    b = tl // NCH
    c0 = sub * CPTS
    iota16 = lax.iota(jnp.int32, 16)
    zero16 = jnp.zeros((16,), jnp.float32)
    lane0 = iota16 == 0

    def zst(i, _):
        cnt_a[pl.ds(i * 16, 16)] = zero16
        sum_a[pl.ds(i * 16, 16)] = zero16
        max_a[pl.ds(i * 16, 16)] = zero16
        return 0
    lax.fori_loop(0, (CPTS + 16) // 16, zst, 0)

    def chunk_body(m, _):
        pltpu.sync_copy(cell_hbm.at[pl.ds((m * 8 + tl) * TP, TP)], cells)
        pltpu.sync_copy(rad_hbm.at[pl.ds((m * 2 + b) * TP, TP)], rads)

        def comp(j, cur):
            v = cells[pl.ds(j * 16, 16)]
            rel = v - c0
            msk = (rel >= 0) & (rel < CPTS)
            pc = plsc.cumsum(msk.astype(jnp.int32))
            dst = cur + pc - 1
            plsc.store_scatter(orel, [dst], rel, mask=msk)
            plsc.store_scatter(orval, [dst], rads[pl.ds(j * 16, 16)],
                               mask=msk)
            return cur + pc[15]
        cnt = lax.fori_loop(0, TP // 16, comp, 0)

        def pt_body(i, _):
            lc = orel[pl.ds(i, 16)][0]
            rv = orval[pl.ds(i, 16)][0]
            rvv = jnp.full((16,), rv, jnp.float32)
            c = cnt_a[pl.ds(lc, 16)]
            cnt_a[pl.ds(lc, 16)] = c + jnp.where(lane0, jnp.float32(1.0),
                                                 jnp.float32(0.0))
            sm = sum_a[pl.ds(lc, 16)]
            sum_a[pl.ds(lc, 16)] = sm + jnp.where(lane0, rvv,
                                                  jnp.float32(0.0))
            mx = max_a[pl.ds(lc, 16)]
            max_a[pl.ds(lc, 16)] = jnp.maximum(
                mx, jnp.where(lane0, rvv, jnp.full((16,), -jnp.inf,
                                                   jnp.float32)))
            return 0
        lax.fori_loop(0, cnt, pt_body, 0)
        return 0
    lax.fori_loop(0, NT, chunk_body, 0)

    def ep(g, _):
        cnt16 = cnt_a[pl.ds(g * 16, 16)]
        sum16 = sum_a[pl.ds(g * 16, 16)]
        max16 = max_a[pl.ds(g * 16, 16)]
        hmean_s[pl.ds(g * 16, 16)] = sum16 / jnp.maximum(cnt16, 1.0)
        hmax_s[pl.ds(g * 16, 16)] = max16
        return 0
    lax.fori_loop(0, CPTS // 16, ep, 0)

    pltpu.sync_copy(hmean_s, hmean_hbm.at[pl.ds(tl * NCELL + c0, CPTS)])
    pltpu.sync_copy(hmax_s, hmax_hbm.at[pl.ds(tl * NCELL + c0, CPTS)])


def _sc_nmax_body(cell_hbm, net_hbm, nmax_hbm,
                  acc, cells, oidx, orel, rows, sem):
    wid = lax.axis_index("s") * 2 + lax.axis_index("c")
    tl = wid // 4
    sub = wid % 4
    b = tl // NCH
    c0 = sub * CPTS
    iota16 = lax.iota(jnp.int32, 16)
    zero16 = jnp.zeros((16,), jnp.float32)
    zero16i = jnp.zeros((16,), jnp.int32)
    for w in range(NWAVE):
        c0 = (w * 4 + sub) * CPT

        def zacc(i, _):
            acc[pl.ds(i * 16, 16)] = zero16
            return 0
        lax.fori_loop(0, CPT * HID // 16, zacc, 0)

        def chunk_body(m, _):
            pltpu.sync_copy(cell_hbm.at[pl.ds((m * 8 + tl) * TP, TP)], cells)
            base = b * T + m * TP

            def comp(j, cur):
                v = cells[pl.ds(j * 16, 16)]
                rel = v - c0
                msk = (rel >= 0) & (rel < CPT)
                pc = plsc.cumsum(msk.astype(jnp.int32))
                dst = cur + pc - 1
                plsc.store_scatter(oidx, [dst],
                                   iota16 + (base + j * 16), mask=msk)
                plsc.store_scatter(orel, [dst], rel, mask=msk)
                return cur + pc[15]
            cnt = lax.fori_loop(0, TP // 16, comp, 0)

            for j in range(GC // 16):
                plsc.store_scatter(oidx, [cnt + j * 16 + iota16],
                                   zero16i + base)

            nsub = (cnt + GC - 1) // GC

            def fire(k):
                pltpu.async_copy(
                    net_hbm.at[oidx.at[pl.ds(k * GC, GC)]],
                    rows.at[pl.ds((k % 2) * GC, GC)], sem)

            def drain(k):
                pltpu.make_async_copy(
                    net_hbm.at[oidx.at[pl.ds(k * GC, GC)]],
                    rows.at[pl.ds((k % 2) * GC, GC)], sem).wait()

            @pl.when(nsub > 0)
            def _():
                fire(0)

            def sub_body(k, _):
                @pl.when(k + 1 < nsub)
                def _():
                    fire(k + 1)
                drain(k)
                npts = jnp.minimum(GC, cnt - k * GC)
                rb = (k % 2) * GC

                def pt_body(i, _):
                    lc = orel[pl.ds(k * GC + i, 16)][0]
                    off = lc * HID
                    for f in range(HID // 16):
                        a = acc[pl.ds(off + f * 16, 16)]
                        r = rows[rb + i, pl.ds(f * 16, 16)]
                        acc[pl.ds(off + f * 16, 16)] = jnp.maximum(a, r)
                    return 0
                lax.fori_loop(0, npts, pt_body, 0)
                return 0
            lax.fori_loop(0, nsub, sub_body, 0)
            return 0
        lax.fori_loop(0, NT, chunk_body, 0)

        pltpu.sync_copy(acc, nmax_hbm.at[pl.ds((tl * NCELL + c0) * HID,
                                               CPT * HID)])


def _scatter(cell3, rad3, net):
    netflat = net.reshape(B * T, HID)
    cellf = cell3.reshape(NT * B * NCH * TP)
    radf = rad3.reshape(NT * B * TP)
    stats_kern = pl.kernel(
        _sc_stats_body,
        out_type=[
            jax.ShapeDtypeStruct((B * NCH * NCELL,), jnp.float32),
            jax.ShapeDtypeStruct((B * NCH * NCELL,), jnp.float32),
        ],
        mesh=plsc.VectorSubcoreMesh(**_SC_MESH),
        compiler_params=pltpu.CompilerParams(needs_layout_passes=False),
        scratch_types=[
            pltpu.VMEM((CPTS + 16,), jnp.float32),
            pltpu.VMEM((CPTS + 16,), jnp.float32),
            pltpu.VMEM((CPTS + 16,), jnp.float32),
            pltpu.VMEM((TP,), jnp.int32),
            pltpu.VMEM((TP,), jnp.float32),
            pltpu.VMEM((OCAP,), jnp.int32),
            pltpu.VMEM((OCAP,), jnp.float32),
            pltpu.VMEM((CPTS,), jnp.float32),
            pltpu.VMEM((CPTS,), jnp.float32),
        ],
    )
    hmean, hmax = stats_kern(cellf, radf)
    nmax_kern = pl.kernel(
        _sc_nmax_body,
        out_type=jax.ShapeDtypeStruct((B * NCH * NCELL * HID,), jnp.float32),
        mesh=plsc.VectorSubcoreMesh(**_SC_MESH),
        compiler_params=pltpu.CompilerParams(needs_layout_passes=False),
        scratch_types=[
            pltpu.VMEM((CPT * HID,), jnp.float32),
            pltpu.VMEM((TP,), jnp.int32),
            pltpu.VMEM((OCAP,), jnp.int32),
            pltpu.VMEM((OCAP,), jnp.int32),
            pltpu.VMEM((2 * GC, HID), jnp.float32),
            pltpu.SemaphoreType.DMA,
        ],
    )
    nmax = nmax_kern(cellf, netflat)
    return (nmax.reshape(B * NCH, NCELL, HID),
            hmean.reshape(B * NCH, NCELL), hmax.reshape(B * NCH, NCELL))


def _conv3(xp_ref, x, w_ref, b_ref, H, W, relu=True):
    Cin = x.shape[-1]
    Co = w_ref.shape[-1]
    xp_ref[...] = jnp.zeros((H + 2, W + 2, Cin), jnp.float32)
    xp_ref[1:H + 1, 1:W + 1, :] = x
    acc = jnp.broadcast_to(b_ref[0], (H * W, Co))
    for dh in range(3):
        for dw in range(3):
            patch = xp_ref[dh:dh + H, dw:dw + W, :].reshape(H * W, Cin)
            acc = acc + jnp.dot(patch, w_ref[dh, dw],
                                preferred_element_type=jnp.float32)
    out = acc.reshape(H, W, Co)
    return jnp.maximum(out, 0.0) if relu else out


def _unet_body(f_ref, w1, b1, w2, b2, w3, b3, w4, b4, w5, b5, w6, b6,
               out_ref, xp1, xp2, xp3, xp4, xp5):
    x = f_ref[0]
    h = _conv3(xp1, x, w1, b1, 64, 64)
    h = _conv3(xp2, h, w2, b2, 64, 64)
    skip = h
    d = jnp.max(h.reshape(32, 2, 64, 64), axis=1)
    d = jnp.max(d.reshape(32, 32, 2, 64), axis=2)
    d = _conv3(xp3, d, w3, b3, 32, 32)
    d = _conv3(xp4, d, w4, b4, 32, 32)
    u = jnp.broadcast_to(d.reshape(32, 1, 32, 128),
                         (32, 2, 32, 128)).reshape(64, 32, 128)
    u = jnp.broadcast_to(u.reshape(64, 32, 1, 128),
                         (64, 32, 2, 128)).reshape(64, 64, 128)
    u = jnp.concatenate([u, skip], axis=2)
    u = _conv3(xp5, u, w5, b5, 64, 64)
    out = jnp.dot(u.reshape(64 * 64, 64), w6[0, 0],
                  preferred_element_type=jnp.float32) + b6[0]
    out_ref[0] = out.reshape(64, 64, CDIM)


def _unet(feat, P):
    def ws(shape):
        return pl.BlockSpec(shape, lambda i: (0,) * len(shape))
    return pl.pallas_call(
        _unet_body,
        grid=(B * NCH,),
        in_specs=[
            pl.BlockSpec((1, 64, 64, HID + 3), lambda i: (i, 0, 0, 0)),
            ws((3, 3, HID + 3, 64)), ws((1, 64)),
            ws((3, 3, 64, 64)), ws((1, 64)),
            ws((3, 3, 64, 128)), ws((1, 128)),
            ws((3, 3, 128, 128)), ws((1, 128)),
            ws((3, 3, 192, 64)), ws((1, 64)),
            ws((1, 1, 64, CDIM)), ws((1, CDIM)),
        ],
        out_specs=pl.BlockSpec((1, 64, 64, CDIM), lambda i: (i, 0, 0, 0)),
        out_shape=jax.ShapeDtypeStruct((B * NCH, 64, 64, CDIM), jnp.float32),
        scratch_shapes=[
            pltpu.VMEM((66, 66, HID + 3), jnp.float32),
            pltpu.VMEM((66, 66, 64), jnp.float32),
            pltpu.VMEM((34, 34, 64), jnp.float32),
            pltpu.VMEM((34, 34, 128), jnp.float32),
            pltpu.VMEM((66, 66, 192), jnp.float32),
        ],
    )(feat, P['u1_W'], P['u1_b'][None], P['u2_W'], P['u2_b'][None],
      P['u3_W'], P['u3_b'][None], P['u4_W'], P['u4_b'][None],
      P['u5_W'], P['u5_b'][None], P['u6_W'], P['u6_b'][None])


def kernel(p, params):
    net = _pointnet(p, params)
    cell, rad = _binning(p)
    nmax, hmean, hmax = _scatter(cell, rad, net)
    feat = jnp.concatenate([
        nmax,
        jnp.zeros((B * NCH, NCELL, 1), jnp.float32),
        hmean[..., None],
        hmax[..., None],
    ], axis=2).reshape(B * NCH, LAT, MER, HID + 3)
    c = _unet(feat, params)
    c = c.reshape(B, NCH, LAT, MER, CDIM)
    centers = jnp.array(_CENTERS, dtype=jnp.float32)
    sphere_centers = jnp.broadcast_to(centers[None], (B, NCH, 3))
    return c, sphere_centers
